# Initial kernel scaffold; baseline (speedup 1.0000x reference)
import jax
import jax.numpy as jnp
from jax.experimental import pallas as pl


def kernel(input, input_embed, W_ih0, W_hh0, b_ih0, b_hh0, W_ih1, W_hh1, b_ih1, b_hh1):
    raise NotImplementedError("write your pallas kernel here")


# speedup vs baseline: 3.1668x; 3.1668x over previous
import jax
import jax.numpy as jnp
from jax.experimental import pallas as pl
from jax.experimental.pallas import tpu as pltpu

EMBED = 512
HIDDEN = 512
BATCH = 64
SEQ = 32
G4 = 4 * HIDDEN
GC = 2 * G4
TS = 4


def _dotT(a, w):
    return jax.lax.dot_general(
        a, w, (((1,), (1,)), ((), ())), preferred_element_type=jnp.float32)


def _xgates_kernel(x_ref, w_ref, b_ref, out_ref):
    x = x_ref[...].reshape(TS * BATCH, EMBED)
    acc = _dotT(x, w_ref[...]) + b_ref[...]
    out_ref[...] = acc.reshape(TS, BATCH, GC)


def _scan_kernel(tok_ref, xg_ref, wh_ref, out_ref, hF_ref, cF_ref, h_scr, c_scr):
    t = pl.program_id(0)

    @pl.when(t == 0)
    def _init():
        h_scr[...] = jnp.zeros_like(h_scr)
        c_scr[...] = jnp.zeros_like(c_scr)

    h = h_scr[...]
    c = c_scr[...]
    g = xg_ref[0] + _dotT(h, wh_ref[...])

    i0 = jax.nn.sigmoid(g[:, 0 * HIDDEN:1 * HIDDEN])
    f0 = jax.nn.sigmoid(g[:, 1 * HIDDEN:2 * HIDDEN])
    g0 = jnp.tanh(g[:, 2 * HIDDEN:3 * HIDDEN])
    o0 = jax.nn.sigmoid(g[:, 3 * HIDDEN:4 * HIDDEN])
    i1 = jax.nn.sigmoid(g[:, 4 * HIDDEN:5 * HIDDEN])
    f1 = jax.nn.sigmoid(g[:, 5 * HIDDEN:6 * HIDDEN])
    g1 = jnp.tanh(g[:, 6 * HIDDEN:7 * HIDDEN])
    o1 = jax.nn.sigmoid(g[:, 7 * HIDDEN:8 * HIDDEN])

    cA = f0 * c + i0 * g0
    hA = o0 * jnp.tanh(cA)
    cB = f1 * c + i1 * g1
    hB = o1 * jnp.tanh(cB)

    m = (tok_ref[0] % 2) == 1
    h_new = jnp.where(m, hB, hA)
    c_new = jnp.where(m, cB, cA)

    h_scr[...] = h_new
    c_scr[...] = c_new
    out_ref[0] = h_new
    hF_ref[...] = h_new
    cF_ref[...] = c_new


def kernel(input, input_embed, W_ih0, W_hh0, b_ih0, b_hh0, W_ih1, W_hh1, b_ih1, b_hh1):
    Wx = jnp.concatenate([W_ih0, W_ih1], axis=0)
    Wh = jnp.concatenate([W_hh0, W_hh1], axis=0)
    b = jnp.concatenate([b_ih0 + b_hh0, b_ih1 + b_hh1]).reshape(1, GC)
    tok = input.T.reshape(SEQ, BATCH, 1)

    xg = pl.pallas_call(
        _xgates_kernel,
        grid=(SEQ // TS,),
        in_specs=[
            pl.BlockSpec((TS, BATCH, EMBED), lambda i: (i, 0, 0)),
            pl.BlockSpec((GC, EMBED), lambda i: (0, 0)),
            pl.BlockSpec((1, GC), lambda i: (0, 0)),
        ],
        out_specs=pl.BlockSpec((TS, BATCH, GC), lambda i: (i, 0, 0)),
        out_shape=jax.ShapeDtypeStruct((SEQ, BATCH, GC), jnp.float32),
    )(input_embed, Wx, b)

    out, hF, cF = pl.pallas_call(
        _scan_kernel,
        grid=(SEQ,),
        in_specs=[
            pl.BlockSpec((1, BATCH, 1), lambda t: (t, 0, 0)),
            pl.BlockSpec((1, BATCH, GC), lambda t: (t, 0, 0)),
            pl.BlockSpec((GC, HIDDEN), lambda t: (0, 0)),
        ],
        out_specs=[
            pl.BlockSpec((1, BATCH, HIDDEN), lambda t: (t, 0, 0)),
            pl.BlockSpec((BATCH, HIDDEN), lambda t: (0, 0)),
            pl.BlockSpec((BATCH, HIDDEN), lambda t: (0, 0)),
        ],
        out_shape=[
            jax.ShapeDtypeStruct((SEQ, BATCH, HIDDEN), jnp.float32),
            jax.ShapeDtypeStruct((BATCH, HIDDEN), jnp.float32),
            jax.ShapeDtypeStruct((BATCH, HIDDEN), jnp.float32),
        ],
        scratch_shapes=[
            pltpu.VMEM((BATCH, HIDDEN), jnp.float32),
            pltpu.VMEM((BATCH, HIDDEN), jnp.float32),
        ],
    )(tok, xg, Wh)

    return out, (hF, cF)


# speedup vs baseline: 3.2745x; 1.0340x over previous
import jax
import jax.numpy as jnp
from jax.experimental import pallas as pl
from jax.experimental.pallas import tpu as pltpu

EMBED = 512
HIDDEN = 512
BATCH = 64
SEQ = 32
G4 = 4 * HIDDEN
GC = 2 * G4
TS = 4


def _dotT(a, w):
    return jax.lax.dot_general(
        a, w, (((1,), (1,)), ((), ())), preferred_element_type=jnp.float32)


def _xgates_kernel(x_ref, w_ref, b_ref, out_ref):
    x = x_ref[...].reshape(TS * BATCH, EMBED)
    acc = _dotT(x, w_ref[...]) + b_ref[...]
    out_ref[...] = acc.reshape(TS, BATCH, GC)


def _scan_kernel(tok_ref, xg_ref, wh_ref, out_ref, hF_ref, cF_ref, h_scr, c_scr):
    t = pl.program_id(0)

    @pl.when(t == 0)
    def _init():
        h_scr[...] = jnp.zeros_like(h_scr)
        c_scr[...] = jnp.zeros_like(c_scr)

    h = h_scr[...]
    c = c_scr[...]
    g = xg_ref[0] + _dotT(h.astype(jnp.bfloat16), wh_ref[...])

    i0 = jax.nn.sigmoid(g[:, 0 * HIDDEN:1 * HIDDEN])
    f0 = jax.nn.sigmoid(g[:, 1 * HIDDEN:2 * HIDDEN])
    g0 = jnp.tanh(g[:, 2 * HIDDEN:3 * HIDDEN])
    o0 = jax.nn.sigmoid(g[:, 3 * HIDDEN:4 * HIDDEN])
    i1 = jax.nn.sigmoid(g[:, 4 * HIDDEN:5 * HIDDEN])
    f1 = jax.nn.sigmoid(g[:, 5 * HIDDEN:6 * HIDDEN])
    g1 = jnp.tanh(g[:, 6 * HIDDEN:7 * HIDDEN])
    o1 = jax.nn.sigmoid(g[:, 7 * HIDDEN:8 * HIDDEN])

    cA = f0 * c + i0 * g0
    hA = o0 * jnp.tanh(cA)
    cB = f1 * c + i1 * g1
    hB = o1 * jnp.tanh(cB)

    m = (tok_ref[0] % 2) == 1
    h_new = jnp.where(m, hB, hA)
    c_new = jnp.where(m, cB, cA)

    h_scr[...] = h_new
    c_scr[...] = c_new
    out_ref[0] = h_new
    hF_ref[...] = h_new
    cF_ref[...] = c_new


def kernel(input, input_embed, W_ih0, W_hh0, b_ih0, b_hh0, W_ih1, W_hh1, b_ih1, b_hh1):
    Wx = jnp.concatenate([W_ih0, W_ih1], axis=0)
    Wh = jnp.concatenate([W_hh0, W_hh1], axis=0).astype(jnp.bfloat16)
    b = jnp.concatenate([b_ih0 + b_hh0, b_ih1 + b_hh1]).reshape(1, GC)
    tok = input.T.reshape(SEQ, BATCH, 1)

    xg = pl.pallas_call(
        _xgates_kernel,
        grid=(SEQ // TS,),
        in_specs=[
            pl.BlockSpec((TS, BATCH, EMBED), lambda i: (i, 0, 0)),
            pl.BlockSpec((GC, EMBED), lambda i: (0, 0)),
            pl.BlockSpec((1, GC), lambda i: (0, 0)),
        ],
        out_specs=pl.BlockSpec((TS, BATCH, GC), lambda i: (i, 0, 0)),
        out_shape=jax.ShapeDtypeStruct((SEQ, BATCH, GC), jnp.float32),
    )(input_embed, Wx, b)

    out, hF, cF = pl.pallas_call(
        _scan_kernel,
        grid=(SEQ,),
        in_specs=[
            pl.BlockSpec((1, BATCH, 1), lambda t: (t, 0, 0)),
            pl.BlockSpec((1, BATCH, GC), lambda t: (t, 0, 0)),
            pl.BlockSpec((GC, HIDDEN), lambda t: (0, 0)),
        ],
        out_specs=[
            pl.BlockSpec((1, BATCH, HIDDEN), lambda t: (t, 0, 0)),
            pl.BlockSpec((BATCH, HIDDEN), lambda t: (0, 0)),
            pl.BlockSpec((BATCH, HIDDEN), lambda t: (0, 0)),
        ],
        out_shape=[
            jax.ShapeDtypeStruct((SEQ, BATCH, HIDDEN), jnp.float32),
            jax.ShapeDtypeStruct((BATCH, HIDDEN), jnp.float32),
            jax.ShapeDtypeStruct((BATCH, HIDDEN), jnp.float32),
        ],
        scratch_shapes=[
            pltpu.VMEM((BATCH, HIDDEN), jnp.float32),
            pltpu.VMEM((BATCH, HIDDEN), jnp.float32),
        ],
    )(tok, xg, Wh)

    return out, (hF, cF)
